# baseline (speedup 1.0000x reference)
import functools

import jax
import jax.numpy as jnp
from jax import lax
from jax.experimental import pallas as pl
from jax.experimental.pallas import tpu as pltpu
from jax.experimental.pallas import tpu_sc as plsc

N_NODES = 10000
D = 128
NC = 2
NS = 16
CHUNK = 128
NLOC = 5120
NACC = 5248
ROWS_PER_TILE = NACC // NS


def _pt_body(psteps, src_hbm, dst_hbm, psrc_out, pdst_out,
             src_v, dst_v, osrc_v, odst_v):
    cid = lax.axis_index("c")
    sid = lax.axis_index("s")
    cap = psteps * CHUNK

    pltpu.sync_copy(src_hbm.at[sid], src_v)
    pltpu.sync_copy(dst_hbm.at[sid], dst_v)

    zeros16i = jnp.zeros((16,), jnp.int32)
    iota16 = lax.iota(jnp.int32, 16)

    def fillo(i, _):
        osrc_v[pl.ds(i * 16, 16)] = zeros16i
        odst_v[pl.ds(i * 16, 16)] = NLOC + (i % 8) * 16 + iota16
        return 0

    lax.fori_loop(0, cap // 16, fillo, 0)

    steps = src_v.shape[0]
    lo = cid * NLOC

    def comp(j, pos):
        for c in range(CHUNK // 16):
            dl = dst_v[j, pl.ds(c * 16, 16)] - lo
            own = (dl >= 0) & (dl < NLOC) & (pos < cap - 16)
            s = src_v[j, pl.ds(c * 16, 16)]
            plsc.store_compressed(osrc_v.at[pl.ds(pos, 16)], s, mask=own)
            plsc.store_compressed(odst_v.at[pl.ds(pos, 16)], dl, mask=own)
            pos = pos + plsc.all_reduce_population_count(own)[0]
        return pos

    lax.fori_loop(0, steps, comp, 0)

    pltpu.sync_copy(osrc_v, psrc_out.at[cid, sid])
    pltpu.sync_copy(odst_v, pdst_out.at[cid, sid])


def _make_sc_partition(steps_per_tile, psteps):
    out_t = jax.ShapeDtypeStruct((NC, NS, psteps * CHUNK), jnp.int32)
    scratch = [
        pltpu.VMEM((steps_per_tile, CHUNK), jnp.int32),
        pltpu.VMEM((steps_per_tile, CHUNK), jnp.int32),
        pltpu.VMEM((psteps * CHUNK,), jnp.int32),
        pltpu.VMEM((psteps * CHUNK,), jnp.int32),
    ]
    mesh = plsc.VectorSubcoreMesh(core_axis_name="c", subcore_axis_name="s")
    return pl.kernel(
        functools.partial(_pt_body, psteps),
        out_type=(out_t, out_t),
        mesh=mesh,
        scratch_types=scratch,
        compiler_params=pltpu.CompilerParams(needs_layout_passes=False),
    )


def _sc_body(with_cnt, localize, y_hbm, src_hbm, dst_hbm, *refs):
    if with_cnt:
        (acc_out, cnt_out, src_v, dst_v, rows_a, rows_b, cnt_v,
         sem_a, sem_b, shared_acc) = refs
    else:
        (acc_out, src_v, dst_v, rows_a, rows_b,
         sem_a, sem_b, shared_acc) = refs
        cnt_out = cnt_v = None

    cid = lax.axis_index("c")
    sid = lax.axis_index("s")

    zeros16 = jnp.zeros((16,), jnp.float32)
    ones16 = jnp.ones((16,), jnp.float32)

    def fill(i, _):
        for c in range(D // 16):
            rows_a[i, pl.ds(c * 16, 16)] = zeros16
        return 0

    lax.fori_loop(0, CHUNK, fill, 0)

    if with_cnt:
        def fillc(i, _):
            cnt_v[pl.ds(i * 16, 16)] = zeros16
            return 0

        lax.fori_loop(0, NACC // 16, fillc, 0)

    base = sid * ROWS_PER_TILE
    rem = ROWS_PER_TILE
    off = 0
    while rem > 0:
        blk = min(CHUNK, rem)
        pltpu.sync_copy(rows_a.at[pl.ds(0, blk)],
                        shared_acc.at[pl.ds(base + off, blk)])
        off += blk
        rem -= blk
    plsc.subcore_barrier()

    if localize:
        pltpu.sync_copy(src_hbm.at[sid], src_v)
        pltpu.sync_copy(dst_hbm.at[sid], dst_v)
    else:
        wid = cid * NS + sid
        pltpu.sync_copy(src_hbm.at[wid], src_v)
        pltpu.sync_copy(dst_hbm.at[wid], dst_v)

    steps = src_v.shape[0]
    lo = cid * NLOC

    if localize:
        def fixdst(j, _):
            for c in range(CHUNK // 16):
                d = dst_v[j, pl.ds(c * 16, 16)] - lo
                ok = (d >= 0) & (d < NLOC)
                dst_v[j, pl.ds(c * 16, 16)] = jnp.where(
                    ok, d, NLOC + (d & (CHUNK - 1)))
            return 0

        lax.fori_loop(0, steps, fixdst, 0)

    def scat(j, rows):
        pltpu.sync_copy(rows, shared_acc.at[dst_v.at[j]], add=True)
        if with_cnt:
            for c in range(CHUNK // 16):
                idx = dst_v[j, pl.ds(c * 16, 16)]
                plsc.addupdate_scatter(cnt_v, [idx], ones16)

    def two(jj, _):
        j0 = 2 * jj
        j1 = j0 + 1
        ca = pltpu.async_copy(y_hbm.at[src_v.at[j0]], rows_a, sem_a)
        cb = pltpu.async_copy(y_hbm.at[src_v.at[j1]], rows_b, sem_b)
        ca.wait()
        scat(j0, rows_a)
        cb.wait()
        scat(j1, rows_b)
        return 0

    lax.fori_loop(0, steps // 2, two, 0)

    if steps % 2:
        j = steps - 1
        pltpu.async_copy(y_hbm.at[src_v.at[j]], rows_a, sem_a).wait()
        scat(j, rows_a)

    plsc.subcore_barrier()

    pltpu.sync_copy(shared_acc.at[pl.ds(base, ROWS_PER_TILE)],
                    acc_out.at[cid, pl.ds(base, ROWS_PER_TILE)])
    if with_cnt:
        pltpu.sync_copy(cnt_v, cnt_out.at[cid, sid])


def _make_sc_aggregate(steps_per_tile, with_cnt, localize=True):
    acc_t = jax.ShapeDtypeStruct((NC, NACC, D), jnp.float32)
    if with_cnt:
        out_type = (acc_t, jax.ShapeDtypeStruct((NC, NS, NACC), jnp.float32))
    else:
        out_type = acc_t
    scratch = [
        pltpu.VMEM((steps_per_tile, CHUNK), jnp.int32),
        pltpu.VMEM((steps_per_tile, CHUNK), jnp.int32),
        pltpu.VMEM((CHUNK, D), jnp.float32),
        pltpu.VMEM((CHUNK, D), jnp.float32),
    ]
    if with_cnt:
        scratch.append(pltpu.VMEM((NACC,), jnp.float32))
    scratch += [
        pltpu.SemaphoreType.DMA,
        pltpu.SemaphoreType.DMA,
        pltpu.VMEM_SHARED((NACC, D), jnp.float32),
    ]
    mesh = plsc.VectorSubcoreMesh(core_axis_name="c", subcore_axis_name="s")
    return pl.kernel(
        functools.partial(_sc_body, with_cnt, localize),
        out_type=out_type,
        mesh=mesh,
        scratch_types=scratch,
        compiler_params=pltpu.CompilerParams(needs_layout_passes=False),
    )


ROW_BLK = 1000


def _copy_body(i_ref, o_ref):
    o_ref[...] = i_ref[...]


def _tc_copy(a):
    return pl.pallas_call(
        _copy_body,
        grid=(a.shape[0],),
        in_specs=[pl.BlockSpec((1,) + a.shape[1:], lambda i: (i, 0, 0))],
        out_specs=pl.BlockSpec((1,) + a.shape[1:], lambda i: (i, 0, 0)),
        out_shape=jax.ShapeDtypeStruct(a.shape, a.dtype),
    )(a)


def _mm2_body(x_ref, wl_ref, wr_ref, y_ref, z_ref):
    xb = x_ref[...]
    dn = (((1,), (1,)), ((), ()))
    y_ref[...] = lax.dot_general(xb, wl_ref[...], dn,
                                 preferred_element_type=jnp.float32)
    z_ref[...] = lax.dot_general(xb, wr_ref[...], dn,
                                 preferred_element_type=jnp.float32)


def _tc_mm2(x, Wl, Wr):
    n = x.shape[0]
    grid = (n // ROW_BLK,)
    bs_row = pl.BlockSpec((ROW_BLK, D), lambda i: (i, 0))
    bs_w = pl.BlockSpec((D, D), lambda i: (0, 0))
    return pl.pallas_call(
        _mm2_body,
        grid=grid,
        in_specs=[bs_row, bs_w, bs_w],
        out_specs=[bs_row, bs_row],
        out_shape=[jax.ShapeDtypeStruct((n, D), jnp.float32)] * 2,
    )(x, Wl, Wr)


def _cnt_body(p_ref, o_ref):
    o_ref[...] = jnp.sum(p_ref[...], axis=1)


def _tc_cnt(parts):
    return pl.pallas_call(
        _cnt_body,
        out_shape=jax.ShapeDtypeStruct((NC, NACC), jnp.float32),
    )(parts)


def _mid_body(a_ref, c_ref, z_ref, bl_ref, wl_ref, wr_ref, y2_ref, z2_ref):
    cnt = c_ref[...]
    inv = 1.0 / jnp.maximum(cnt, 1.0)
    h = jnp.maximum(a_ref[...] * inv + bl_ref[...] + z_ref[...], 0.0)
    dn = (((1,), (1,)), ((), ()))
    y2_ref[...] = lax.dot_general(h, wl_ref[...], dn,
                                  preferred_element_type=jnp.float32)
    z2_ref[...] = lax.dot_general(h, wr_ref[...], dn,
                                  preferred_element_type=jnp.float32)


def _tc_mid(agg, cnt, z1, bl1, Wl2, Wr2):
    n = z1.shape[0]
    grid = (n // ROW_BLK,)
    bs_row = pl.BlockSpec((ROW_BLK, D), lambda i: (i, 0))
    bs_cnt = pl.BlockSpec((ROW_BLK, 1), lambda i: (i, 0))
    bs_w = pl.BlockSpec((D, D), lambda i: (0, 0))
    bs_b = pl.BlockSpec((1, D), lambda i: (0, 0))
    return pl.pallas_call(
        _mid_body,
        grid=grid,
        in_specs=[bs_row, bs_cnt, bs_row, bs_b, bs_w, bs_w],
        out_specs=[bs_row, bs_row],
        out_shape=[jax.ShapeDtypeStruct((n, D), jnp.float32)] * 2,
    )(agg, cnt, z1, bl1, Wl2, Wr2)


def _fin_body(a_ref, c_ref, z_ref, bl_ref, out_ref):
    cnt = c_ref[...]
    inv = 1.0 / jnp.maximum(cnt, 1.0)
    out_ref[...] = a_ref[...] * inv + bl_ref[...] + z_ref[...]


def _tc_fin(agg, cnt, z2, bl2):
    n = z2.shape[0]
    grid = (n // ROW_BLK,)
    bs_row = pl.BlockSpec((ROW_BLK, D), lambda i: (i, 0))
    bs_cnt = pl.BlockSpec((ROW_BLK, 1), lambda i: (i, 0))
    bs_b = pl.BlockSpec((1, D), lambda i: (0, 0))
    return pl.pallas_call(
        _fin_body,
        grid=grid,
        in_specs=[bs_row, bs_cnt, bs_row, bs_b],
        out_specs=bs_row,
        out_shape=jax.ShapeDtypeStruct((n, D), jnp.float32),
    )(agg, cnt, z2, bl2)


def _assemble(parts, n):
    return jnp.concatenate([parts[0, :NLOC], parts[1, :n - NLOC]], axis=0)


def kernel(x, edge_index, Wl1, bl1, Wr1, Wl2, bl2, Wr2):
    n, d = x.shape
    e = edge_index.shape[1]
    src = edge_index[0].astype(jnp.int32)
    dst = edge_index[1].astype(jnp.int32)

    steps = -(-e // (NS * CHUNK))
    e_pad = steps * NS * CHUNK
    src_p = jnp.concatenate(
        [src, jnp.zeros((e_pad - e,), jnp.int32)]
    ).reshape(steps * CHUNK, NS).T.reshape(NS, steps, CHUNK)
    dst_p = jnp.concatenate(
        [dst, jnp.full((e_pad - e,), n, jnp.int32)]
    ).reshape(steps * CHUNK, NS).T.reshape(NS, steps, CHUNK)

    psteps = -(-(int(0.512 * steps * CHUNK) + (e_pad - e) // NS
                 + 10 * int((steps * CHUNK) ** 0.5) + 128) // CHUNK)
    psteps += (-psteps) % 8

    sc_part = _make_sc_partition(steps, psteps)
    sc_agg_cnt = _make_sc_aggregate(psteps, with_cnt=True, localize=False)
    sc_agg = _make_sc_aggregate(psteps, with_cnt=False, localize=False)

    psrc, pdst = sc_part(src_p, dst_p)
    psrc = _tc_copy(psrc.reshape(NC * NS, psteps, CHUNK))
    pdst = _tc_copy(pdst.reshape(NC * NS, psteps, CHUNK))

    y1, z1 = _tc_mm2(x, Wl1, Wr1)
    acc1, cnt_parts = sc_agg_cnt(y1, psrc, pdst)
    agg1 = _assemble(acc1, n)
    cnt_red = _tc_cnt(cnt_parts)
    cnt = jnp.concatenate([cnt_red[0, :NLOC], cnt_red[1, :n - NLOC]])[:, None]
    y2, z2 = _tc_mid(agg1, cnt, z1, bl1.reshape(1, D), Wl2, Wr2)
    acc2 = sc_agg(y2, psrc, pdst)
    agg2 = _assemble(acc2, n)
    out = _tc_fin(agg2, cnt, z2, bl2.reshape(1, D))
    return out


# speedup vs baseline: 7.0545x; 7.0545x over previous
import functools

import jax
import jax.numpy as jnp
from jax import lax
from jax.experimental import pallas as pl
from jax.experimental.pallas import tpu as pltpu
from jax.experimental.pallas import tpu_sc as plsc

N_NODES = 10000
D = 128
NC = 2
NS = 16
CHUNK = 128
NLOC = 5120
NACC = 5248
ROWS_PER_TILE = NACC // NS


def _sc_body(with_cnt, y_hbm, src_hbm, dst_hbm, *refs):
    if with_cnt:
        (acc_out, cnt_out, src_v, dst_v, rows_a, rows_b, cnt_v,
         sem_a, sem_b, shared_acc) = refs
    else:
        (acc_out, src_v, dst_v, rows_a, rows_b,
         sem_a, sem_b, shared_acc) = refs
        cnt_out = cnt_v = None

    cid = lax.axis_index("c")
    sid = lax.axis_index("s")

    zeros16 = jnp.zeros((16,), jnp.float32)
    ones16 = jnp.ones((16,), jnp.float32)

    def fill(i, _):
        for c in range(D // 16):
            rows_a[i, pl.ds(c * 16, 16)] = zeros16
        return 0

    lax.fori_loop(0, CHUNK, fill, 0)

    if with_cnt:
        def fillc(i, _):
            cnt_v[pl.ds(i * 16, 16)] = zeros16
            return 0

        lax.fori_loop(0, NACC // 16, fillc, 0)

    base = sid * ROWS_PER_TILE
    rem = ROWS_PER_TILE
    off = 0
    while rem > 0:
        blk = min(CHUNK, rem)
        pltpu.sync_copy(rows_a.at[pl.ds(0, blk)],
                        shared_acc.at[pl.ds(base + off, blk)])
        off += blk
        rem -= blk
    plsc.subcore_barrier()

    pltpu.sync_copy(src_hbm.at[sid], src_v)
    pltpu.sync_copy(dst_hbm.at[sid], dst_v)

    steps = src_v.shape[0]
    lo = cid * NLOC

    def fixdst(j, _):
        for c in range(CHUNK // 16):
            d = dst_v[j, pl.ds(c * 16, 16)] - lo
            ok = (d >= 0) & (d < NLOC)
            dst_v[j, pl.ds(c * 16, 16)] = jnp.where(
                ok, d, NLOC + (d & (CHUNK - 1)))
        return 0

    lax.fori_loop(0, steps, fixdst, 0)

    def scat(j, rows):
        pltpu.sync_copy(rows, shared_acc.at[dst_v.at[j]], add=True)
        if with_cnt:
            for c in range(CHUNK // 16):
                idx = dst_v[j, pl.ds(c * 16, 16)]
                plsc.addupdate_scatter(cnt_v, [idx], ones16)

    def two(jj, _):
        j0 = 2 * jj
        j1 = j0 + 1
        ca = pltpu.async_copy(y_hbm.at[src_v.at[j0]], rows_a, sem_a)
        cb = pltpu.async_copy(y_hbm.at[src_v.at[j1]], rows_b, sem_b)
        ca.wait()
        scat(j0, rows_a)
        cb.wait()
        scat(j1, rows_b)
        return 0

    lax.fori_loop(0, steps // 2, two, 0)

    if steps % 2:
        j = steps - 1
        pltpu.async_copy(y_hbm.at[src_v.at[j]], rows_a, sem_a).wait()
        scat(j, rows_a)

    plsc.subcore_barrier()

    pltpu.sync_copy(shared_acc.at[pl.ds(base, ROWS_PER_TILE)],
                    acc_out.at[cid, pl.ds(base, ROWS_PER_TILE)])
    if with_cnt:
        pltpu.sync_copy(cnt_v, cnt_out.at[cid, sid])


def _make_sc_aggregate(steps_per_tile, with_cnt):
    acc_t = jax.ShapeDtypeStruct((NC, NACC, D), jnp.float32)
    if with_cnt:
        out_type = (acc_t, jax.ShapeDtypeStruct((NC, NS, NACC), jnp.float32))
    else:
        out_type = acc_t
    scratch = [
        pltpu.VMEM((steps_per_tile, CHUNK), jnp.int32),
        pltpu.VMEM((steps_per_tile, CHUNK), jnp.int32),
        pltpu.VMEM((CHUNK, D), jnp.float32),
        pltpu.VMEM((CHUNK, D), jnp.float32),
    ]
    if with_cnt:
        scratch.append(pltpu.VMEM((NACC,), jnp.float32))
    scratch += [
        pltpu.SemaphoreType.DMA,
        pltpu.SemaphoreType.DMA,
        pltpu.VMEM_SHARED((NACC, D), jnp.float32),
    ]
    mesh = plsc.VectorSubcoreMesh(core_axis_name="c", subcore_axis_name="s")
    return pl.kernel(
        functools.partial(_sc_body, with_cnt),
        out_type=out_type,
        mesh=mesh,
        scratch_types=scratch,
        compiler_params=pltpu.CompilerParams(needs_layout_passes=False),
    )


ROW_BLK = 1000


def _mm2_body(x_ref, wl_ref, wr_ref, y_ref, z_ref):
    xb = x_ref[...]
    dn = (((1,), (1,)), ((), ()))
    y_ref[...] = lax.dot_general(xb, wl_ref[...], dn,
                                 preferred_element_type=jnp.float32)
    z_ref[...] = lax.dot_general(xb, wr_ref[...], dn,
                                 preferred_element_type=jnp.float32)


def _tc_mm2(x, Wl, Wr):
    n = x.shape[0]
    grid = (n // ROW_BLK,)
    bs_row = pl.BlockSpec((ROW_BLK, D), lambda i: (i, 0))
    bs_w = pl.BlockSpec((D, D), lambda i: (0, 0))
    return pl.pallas_call(
        _mm2_body,
        grid=grid,
        in_specs=[bs_row, bs_w, bs_w],
        out_specs=[bs_row, bs_row],
        out_shape=[jax.ShapeDtypeStruct((n, D), jnp.float32)] * 2,
    )(x, Wl, Wr)


def _cnt_body(p_ref, o_ref):
    o_ref[...] = jnp.sum(p_ref[...], axis=1)


def _tc_cnt(parts):
    return pl.pallas_call(
        _cnt_body,
        out_shape=jax.ShapeDtypeStruct((NC, NACC), jnp.float32),
    )(parts)


def _mid_body(a_ref, c_ref, z_ref, bl_ref, wl_ref, wr_ref, y2_ref, z2_ref):
    cnt = c_ref[...]
    inv = 1.0 / jnp.maximum(cnt, 1.0)
    h = jnp.maximum(a_ref[...] * inv + bl_ref[...] + z_ref[...], 0.0)
    dn = (((1,), (1,)), ((), ()))
    y2_ref[...] = lax.dot_general(h, wl_ref[...], dn,
                                  preferred_element_type=jnp.float32)
    z2_ref[...] = lax.dot_general(h, wr_ref[...], dn,
                                  preferred_element_type=jnp.float32)


def _tc_mid(agg, cnt, z1, bl1, Wl2, Wr2):
    n = z1.shape[0]
    grid = (n // ROW_BLK,)
    bs_row = pl.BlockSpec((ROW_BLK, D), lambda i: (i, 0))
    bs_cnt = pl.BlockSpec((ROW_BLK, 1), lambda i: (i, 0))
    bs_w = pl.BlockSpec((D, D), lambda i: (0, 0))
    bs_b = pl.BlockSpec((1, D), lambda i: (0, 0))
    return pl.pallas_call(
        _mid_body,
        grid=grid,
        in_specs=[bs_row, bs_cnt, bs_row, bs_b, bs_w, bs_w],
        out_specs=[bs_row, bs_row],
        out_shape=[jax.ShapeDtypeStruct((n, D), jnp.float32)] * 2,
    )(agg, cnt, z1, bl1, Wl2, Wr2)


def _fin_body(a_ref, c_ref, z_ref, bl_ref, out_ref):
    cnt = c_ref[...]
    inv = 1.0 / jnp.maximum(cnt, 1.0)
    out_ref[...] = a_ref[...] * inv + bl_ref[...] + z_ref[...]


def _tc_fin(agg, cnt, z2, bl2):
    n = z2.shape[0]
    grid = (n // ROW_BLK,)
    bs_row = pl.BlockSpec((ROW_BLK, D), lambda i: (i, 0))
    bs_cnt = pl.BlockSpec((ROW_BLK, 1), lambda i: (i, 0))
    bs_b = pl.BlockSpec((1, D), lambda i: (0, 0))
    return pl.pallas_call(
        _fin_body,
        grid=grid,
        in_specs=[bs_row, bs_cnt, bs_row, bs_b],
        out_specs=bs_row,
        out_shape=jax.ShapeDtypeStruct((n, D), jnp.float32),
    )(agg, cnt, z2, bl2)


def _assemble(parts, n):
    return jnp.concatenate([parts[0, :NLOC], parts[1, :n - NLOC]], axis=0)


def kernel(x, edge_index, Wl1, bl1, Wr1, Wl2, bl2, Wr2):
    n, d = x.shape
    e = edge_index.shape[1]
    src = edge_index[0].astype(jnp.int32)
    dst = edge_index[1].astype(jnp.int32)

    steps = -(-e // (NS * CHUNK))
    e_pad = steps * NS * CHUNK
    src_p = jnp.concatenate(
        [src, jnp.zeros((e_pad - e,), jnp.int32)]).reshape(NS, steps, CHUNK)
    dst_p = jnp.concatenate(
        [dst, jnp.full((e_pad - e,), n, jnp.int32)]).reshape(NS, steps, CHUNK)

    sc_agg_cnt = _make_sc_aggregate(steps, with_cnt=True)
    sc_agg = _make_sc_aggregate(steps, with_cnt=False)

    y1, z1 = _tc_mm2(x, Wl1, Wr1)
    acc1, cnt_parts = sc_agg_cnt(y1, src_p, dst_p)
    agg1 = _assemble(acc1, n)
    cnt_red = _tc_cnt(cnt_parts)
    cnt = jnp.concatenate([cnt_red[0, :NLOC], cnt_red[1, :n - NLOC]])[:, None]
    y2, z2 = _tc_mid(agg1, cnt, z1, bl1.reshape(1, D), Wl2, Wr2)
    acc2 = sc_agg(y2, src_p, dst_p)
    agg2 = _assemble(acc2, n)
    out = _tc_fin(agg2, cnt, z2, bl2.reshape(1, D))
    return out
